# Initial kernel scaffold; baseline (speedup 1.0000x reference)
import jax
import jax.numpy as jnp
from jax.experimental import pallas as pl


def kernel(node_input, node_attr, edge_src, edge_dst, edge_attr, edge_scalars, W_sc, W_lin1, W_fc1, W_fc2, W_lin2):
    raise NotImplementedError("write your pallas kernel here")


# speedup vs baseline: 2.2846x; 2.2846x over previous
import functools

import jax
import jax.numpy as jnp
import numpy as np
from jax import lax
from jax.experimental import pallas as pl
from jax.experimental.pallas import tpu as pltpu
from jax.experimental.pallas import tpu_sc as plsc

N = 10000
E = 320000
D = 128
S = 16
H = 32
NUM_NEIGHBORS = 32.0
_SILU_CST = 1.679177

_C_S = float(np.sin(np.pi / 8.0))
_C_X = float(np.cos(np.pi / 8.0))


_BE = 3200


def _edge_mlp_body(es_ref, attr_ref, wfc1_ref, wfc2_ref, out_ref):
    es = es_ref[...]
    h = jnp.dot(es, wfc1_ref[...], preferred_element_type=jnp.float32)
    h = h * (1.0 / np.sqrt(float(S)))
    h = (h * jax.nn.sigmoid(h)) * _SILU_CST
    w = jnp.dot(h, wfc2_ref[...], preferred_element_type=jnp.float32)
    out_ref[...] = w * (attr_ref[...] * (1.0 / np.sqrt(float(H))))


def _edge_mlp(edge_scalars, edge_attr, W_fc1, W_fc2):
    grid = (E // _BE,)
    return pl.pallas_call(
        _edge_mlp_body,
        grid=grid,
        in_specs=[
            pl.BlockSpec((_BE, S), lambda i: (i, 0)),
            pl.BlockSpec((_BE, 1), lambda i: (i, 0)),
            pl.BlockSpec((S, H), lambda i: (0, 0)),
            pl.BlockSpec((H, D), lambda i: (0, 0)),
        ],
        out_specs=pl.BlockSpec((_BE, D), lambda i: (i, 0)),
        out_shape=jax.ShapeDtypeStruct((E, D), jnp.float32),
    )(edge_scalars, edge_attr, W_fc1, W_fc2)


_BN = 2000


def _node_pre_body(x_ref, attr_ref, wsc_ref, wlin1_ref, s_ref, xl_ref):
    xa = x_ref[...] * attr_ref[...]
    s_ref[...] = jnp.dot(xa, wsc_ref[...], preferred_element_type=jnp.float32) * (
        _C_S / np.sqrt(float(D))
    )
    xl_ref[...] = jnp.dot(xa, wlin1_ref[...], preferred_element_type=jnp.float32) * (
        1.0 / np.sqrt(float(D))
    )


def _node_pre(node_input, node_attr, W_sc, W_lin1):
    grid = (N // _BN,)
    return pl.pallas_call(
        _node_pre_body,
        grid=grid,
        in_specs=[
            pl.BlockSpec((_BN, D), lambda i: (i, 0)),
            pl.BlockSpec((_BN, 1), lambda i: (i, 0)),
            pl.BlockSpec((D, D), lambda i: (0, 0)),
            pl.BlockSpec((D, D), lambda i: (0, 0)),
        ],
        out_specs=[
            pl.BlockSpec((_BN, D), lambda i: (i, 0)),
            pl.BlockSpec((_BN, D), lambda i: (i, 0)),
        ],
        out_shape=[
            jax.ShapeDtypeStruct((N, D), jnp.float32),
            jax.ShapeDtypeStruct((N, D), jnp.float32),
        ],
    )(node_input, node_attr, W_sc, W_lin1)


_NC = 2
_NS = 16
_EPT = E // (_NC * _NS)
_C = 80
_ITERS = _EPT // _C
_NP = 10240
_RPT = _NP // _NS
_ZR = 128


def _sc_scatter_body(
    xl_hbm, wa_hbm, src_hbm, dst_hbm, out_hbm, idx_src, idx_dst, rows, wab, zbuf, acc, sem
):
    cid = lax.axis_index("c")
    sid = lax.axis_index("s")

    zero16 = jnp.zeros((16,), jnp.float32)

    def zrow(i, carry):
        for v in range(D // 16):
            zbuf[i, pl.ds(v * 16, 16)] = zero16
        return carry

    lax.fori_loop(0, _ZR, zrow, 0)

    def zcopy(j, carry):
        pltpu.sync_copy(zbuf, acc.at[pl.ds(sid * _RPT + j * _ZR, _ZR)])
        return carry

    lax.fori_loop(0, _RPT // _ZR, zcopy, 0)

    plsc.subcore_barrier()

    ebase = cid * (E // _NC) + sid * _EPT

    def step(i, carry):
        base = ebase + i * _C
        pltpu.sync_copy(src_hbm.at[pl.ds(base, _C)], idx_src)
        pltpu.sync_copy(dst_hbm.at[pl.ds(base, _C)], idx_dst)
        pltpu.async_copy(xl_hbm.at[idx_src], rows, sem).wait()
        pltpu.sync_copy(wa_hbm.at[pl.ds(base, _C)], wab)

        def erow(e, c2):
            for v in range(D // 16):
                sl = pl.ds(v * 16, 16)
                rows[e, sl] = rows[e, sl] * wab[e, sl]
            return c2

        lax.fori_loop(0, _C, erow, 0)
        pltpu.sync_copy(rows, acc.at[idx_dst], add=True)
        return carry

    lax.fori_loop(0, _ITERS, step, 0)

    plsc.subcore_barrier()
    pltpu.sync_copy(
        acc.at[pl.ds(sid * _RPT, _RPT)], out_hbm.at[cid, pl.ds(sid * _RPT, _RPT)]
    )


def _sc_scatter(xl, wa, edge_src, edge_dst):
    mesh = plsc.VectorSubcoreMesh(core_axis_name="c", subcore_axis_name="s")
    kern = functools.partial(
        pl.kernel,
        out_type=jax.ShapeDtypeStruct((_NC, _NP, D), jnp.float32),
        mesh=mesh,
        scratch_types=[
            pltpu.VMEM((_C,), jnp.int32),
            pltpu.VMEM((_C,), jnp.int32),
            pltpu.VMEM((_C, D), jnp.float32),
            pltpu.VMEM((_C, D), jnp.float32),
            pltpu.VMEM((_ZR, D), jnp.float32),
            pltpu.VMEM_SHARED((_NP, D), jnp.float32),
            pltpu.SemaphoreType.DMA,
        ],
    )(_sc_scatter_body)
    return kern(xl, wa, edge_src, edge_dst)[:, :N, :]


def _post_body(s_ref, parts_ref, attr_ref, wlin2_ref, out_ref):
    agg = (parts_ref[0] + parts_ref[1]) * attr_ref[...]
    x2 = jnp.dot(agg, wlin2_ref[...], preferred_element_type=jnp.float32)
    out_ref[...] = s_ref[...] + x2 * (_C_X / (np.sqrt(NUM_NEIGHBORS) * np.sqrt(float(D))))


def _post(s_pre, parts, node_attr, W_lin2):
    grid = (N // _BN,)
    return pl.pallas_call(
        _post_body,
        grid=grid,
        in_specs=[
            pl.BlockSpec((_BN, D), lambda i: (i, 0)),
            pl.BlockSpec((_NC, _BN, D), lambda i: (0, i, 0)),
            pl.BlockSpec((_BN, 1), lambda i: (i, 0)),
            pl.BlockSpec((D, D), lambda i: (0, 0)),
        ],
        out_specs=pl.BlockSpec((_BN, D), lambda i: (i, 0)),
        out_shape=jax.ShapeDtypeStruct((N, D), jnp.float32),
    )(s_pre, parts, node_attr, W_lin2)


def kernel(node_input, node_attr, edge_src, edge_dst, edge_attr, edge_scalars,
           W_sc, W_lin1, W_fc1, W_fc2, W_lin2):
    wa = _edge_mlp(edge_scalars, edge_attr, W_fc1, W_fc2)
    s_pre, xl = _node_pre(node_input, node_attr, W_sc, W_lin1)
    parts = _sc_scatter(xl, wa, edge_src, edge_dst)
    return _post(s_pre, parts, node_attr, W_lin2)


# speedup vs baseline: 2.5076x; 1.0976x over previous
import functools

import jax
import jax.numpy as jnp
import numpy as np
from jax import lax
from jax.experimental import pallas as pl
from jax.experimental.pallas import tpu as pltpu
from jax.experimental.pallas import tpu_sc as plsc

N = 10000
E = 320000
D = 128
S = 16
H = 32
NUM_NEIGHBORS = 32.0
_SILU_CST = 1.679177

_C_S = float(np.sin(np.pi / 8.0))
_C_X = float(np.cos(np.pi / 8.0))


_EPAD = E + 1280
_BE = 1280


def _edge_mlp_body(es_ref, attr_ref, wfc1_ref, wfc2_ref, out_ref):
    es = es_ref[...]
    h = jnp.dot(es, wfc1_ref[...], preferred_element_type=jnp.float32)
    h = h * (1.0 / np.sqrt(float(S)))
    h = (h * jax.nn.sigmoid(h)) * _SILU_CST
    w = jnp.dot(h, wfc2_ref[...], preferred_element_type=jnp.float32)
    out_ref[...] = w * (attr_ref[...] * (1.0 / np.sqrt(float(H))))


def _edge_mlp(edge_scalars, edge_attr, W_fc1, W_fc2):
    zs = jnp.zeros((_EPAD - E, S), dtype=edge_scalars.dtype)
    za = jnp.zeros((_EPAD - E, 1), dtype=edge_attr.dtype)
    edge_scalars = jnp.concatenate([edge_scalars, zs])
    edge_attr = jnp.concatenate([edge_attr, za])
    grid = (_EPAD // _BE,)
    return pl.pallas_call(
        _edge_mlp_body,
        grid=grid,
        in_specs=[
            pl.BlockSpec((_BE, S), lambda i: (i, 0)),
            pl.BlockSpec((_BE, 1), lambda i: (i, 0)),
            pl.BlockSpec((S, H), lambda i: (0, 0)),
            pl.BlockSpec((H, D), lambda i: (0, 0)),
        ],
        out_specs=pl.BlockSpec((_BE, D), lambda i: (i, 0)),
        out_shape=jax.ShapeDtypeStruct((_EPAD, D), jnp.float32),
    )(edge_scalars, edge_attr, W_fc1, W_fc2)


_BN = 2000


def _node_pre_body(x_ref, attr_ref, wsc_ref, wlin1_ref, s_ref, xl_ref):
    xa = x_ref[...] * attr_ref[...]
    s_ref[...] = jnp.dot(xa, wsc_ref[...], preferred_element_type=jnp.float32) * (
        _C_S / np.sqrt(float(D))
    )
    xl_ref[...] = jnp.dot(xa, wlin1_ref[...], preferred_element_type=jnp.float32) * (
        1.0 / np.sqrt(float(D))
    )


def _node_pre(node_input, node_attr, W_sc, W_lin1):
    grid = (N // _BN,)
    return pl.pallas_call(
        _node_pre_body,
        grid=grid,
        in_specs=[
            pl.BlockSpec((_BN, D), lambda i: (i, 0)),
            pl.BlockSpec((_BN, 1), lambda i: (i, 0)),
            pl.BlockSpec((D, D), lambda i: (0, 0)),
            pl.BlockSpec((D, D), lambda i: (0, 0)),
        ],
        out_specs=[
            pl.BlockSpec((_BN, D), lambda i: (i, 0)),
            pl.BlockSpec((_BN, D), lambda i: (i, 0)),
        ],
        out_shape=[
            jax.ShapeDtypeStruct((N, D), jnp.float32),
            jax.ShapeDtypeStruct((N, D), jnp.float32),
        ],
    )(node_input, node_attr, W_sc, W_lin1)


_NC = 2
_NS = 16
_C = 80
_EALIGN = _NS * _C
_MAXEPT = E // _NS
_NH = N // 2
_AP = 5120
_TRASH = 5040
_RPT = _AP // _NS


def _sc_scatter_body(xl_hbm, wa_hbm, src_hbm, dst_hbm, es_hbm, out_hbm,
                     src_v, dst_v, es_v, idxd0, idxd1, sidx, rows0, rows1,
                     wab0, wab1, acc, sg0, sg1, sw0, sw1):
    cid = lax.axis_index("c")
    sid = lax.axis_index("s")

    zero16 = jnp.zeros((16,), jnp.float32)

    def zrow(i, carry):
        for v in range(D // 16):
            rows0[i, pl.ds(v * 16, 16)] = zero16
        return carry

    lax.fori_loop(0, _C, zrow, 0)

    def zcopy(j, carry):
        off = pl.multiple_of(sid * _RPT + j * _C, _C)
        pltpu.sync_copy(rows0, acc.at[pl.ds(off, _C)])
        return carry

    lax.fori_loop(0, _RPT // _C, zcopy, 0)

    pltpu.sync_copy(es_hbm, es_v)
    es = es_v[...][0]
    ept = jnp.where(cid == 0, es, E - es) // _NS
    iters = ept // _C
    ebase = pl.multiple_of(jnp.where(cid == 0, 0, es) + sid * ept, _C)

    pltpu.sync_copy(src_hbm.at[pl.ds(ebase, _MAXEPT)], src_v)
    pltpu.sync_copy(dst_hbm.at[pl.ds(ebase, _MAXEPT)], dst_v)

    plsc.subcore_barrier()

    rows = (rows0, rows1)
    wabs = (wab0, wab1)
    idxds = (idxd0, idxd1)
    sgs = (sg0, sg1)
    sws = (sw0, sw1)

    def issue(k, b):
        koff = pl.multiple_of(k * _C, _C)
        eoff = pl.multiple_of(ebase + k * _C, _C)
        pltpu.async_copy(xl_hbm.at[src_v.at[pl.ds(koff, _C)]], rows[b], sgs[b])
        pltpu.async_copy(wa_hbm.at[pl.ds(eoff, _C)], wabs[b], sws[b])

    def process(k, b):
        koff = pl.multiple_of(k * _C, _C)
        eoff = pl.multiple_of(ebase + k * _C, _C)
        pltpu.make_async_copy(
            xl_hbm.at[src_v.at[pl.ds(koff, _C)]], rows[b], sgs[b]
        ).wait()
        pltpu.make_async_copy(
            wa_hbm.at[pl.ds(eoff, _C)], wabs[b], sws[b]
        ).wait()

        def erow(e, c2):
            for v in range(D // 16):
                sl = pl.ds(v * 16, 16)
                rows[b][e, sl] = rows[b][e, sl] * wabs[b][e, sl]
            return c2

        lax.fori_loop(0, _C, erow, 0)
        koff16 = pl.multiple_of(k * _C, 16)

        @pl.when(cid == 0)
        def _():
            for j in range(_C // 16):
                idxds[b][pl.ds(j * 16, 16)] = dst_v[pl.ds(koff16 + j * 16, 16)]

        @pl.when(cid == 1)
        def _():
            for j in range(_C // 16):
                d16 = dst_v[pl.ds(koff16 + j * 16, 16)]
                idxds[b][pl.ds(j * 16, 16)] = jnp.where(
                    d16 < _NH, _TRASH, d16 - _NH
                )

        pltpu.sync_copy(rows[b], acc.at[idxds[b]], add=True)

    @pl.when(iters > 0)
    def _():
        issue(0, 0)

    def outer(o, carry):
        k0 = 2 * o
        k1 = 2 * o + 1

        @pl.when(k1 < iters)
        def _():
            issue(k1, 1)

        process(k0, 0)

        @pl.when(k0 + 2 < iters)
        def _():
            issue(k0 + 2, 0)

        @pl.when(k1 < iters)
        def _():
            process(k1, 1)

        return carry

    lax.fori_loop(0, (iters + 1) // 2, outer, 0)

    @pl.when(cid == 0)
    def _():
        sb = pl.multiple_of(es + sid * _C, _C)
        pltpu.sync_copy(src_hbm.at[pl.ds(sb, _C)], sidx)
        pltpu.async_copy(xl_hbm.at[sidx], rows0, sg0)
        pltpu.sync_copy(wa_hbm.at[pl.ds(sb, _C)], wab0)
        pltpu.make_async_copy(xl_hbm.at[sidx], rows0, sg0).wait()

        def srow(e, c2):
            for v in range(D // 16):
                sl = pl.ds(v * 16, 16)
                rows0[e, sl] = rows0[e, sl] * wab0[e, sl]
            return c2

        lax.fori_loop(0, _C, srow, 0)
        pltpu.sync_copy(dst_hbm.at[pl.ds(sb, _C)], sidx)
        for j in range(_C // 16):
            d16 = sidx[pl.ds(j * 16, 16)]
            idxd0[pl.ds(j * 16, 16)] = jnp.where(d16 < _NH, d16, _TRASH)
        pltpu.sync_copy(rows0, acc.at[idxd0], add=True)

    plsc.subcore_barrier()
    ooff = pl.multiple_of(sid * _RPT, _RPT)
    pltpu.sync_copy(
        acc.at[pl.ds(ooff, _RPT)], out_hbm.at[cid, pl.ds(ooff, _RPT)]
    )


def _sc_scatter(xl, wa, edge_src, edge_dst):
    ec0 = jnp.searchsorted(edge_dst, _NH).astype(jnp.int32)
    es = (ec0 // _EALIGN) * _EALIGN
    es_arr = jnp.full((16,), es, dtype=jnp.int32)
    pad = jnp.zeros((_MAXEPT,), dtype=jnp.int32)
    src_p = jnp.concatenate([edge_src, pad])
    dst_p = jnp.concatenate([edge_dst, pad])
    mesh = plsc.VectorSubcoreMesh(core_axis_name="c", subcore_axis_name="s")
    kern = functools.partial(
        pl.kernel,
        out_type=jax.ShapeDtypeStruct((_NC, _AP, D), jnp.float32),
        mesh=mesh,
        scratch_types=[
            pltpu.VMEM((_MAXEPT,), jnp.int32),
            pltpu.VMEM((_MAXEPT,), jnp.int32),
            pltpu.VMEM((16,), jnp.int32),
            pltpu.VMEM((_C,), jnp.int32),
            pltpu.VMEM((_C,), jnp.int32),
            pltpu.VMEM((_C,), jnp.int32),
            pltpu.VMEM((_C, D), jnp.float32),
            pltpu.VMEM((_C, D), jnp.float32),
            pltpu.VMEM((_C, D), jnp.float32),
            pltpu.VMEM((_C, D), jnp.float32),
            pltpu.VMEM_SHARED((_AP, D), jnp.float32),
            pltpu.SemaphoreType.DMA,
            pltpu.SemaphoreType.DMA,
            pltpu.SemaphoreType.DMA,
            pltpu.SemaphoreType.DMA,
        ],
    )(_sc_scatter_body)
    return kern(xl, wa, src_p, dst_p, es_arr)


def _post_body(s_ref, parts_ref, attr_ref, wlin2_ref, out_ref):
    agg = parts_ref[0] * attr_ref[...]
    x2 = jnp.dot(agg, wlin2_ref[...], preferred_element_type=jnp.float32)
    out_ref[...] = s_ref[...] + x2 * (_C_X / (np.sqrt(NUM_NEIGHBORS) * np.sqrt(float(D))))


def _post(s_pre, parts, node_attr, W_lin2):
    grid = (N // _NH,)
    return pl.pallas_call(
        _post_body,
        grid=grid,
        in_specs=[
            pl.BlockSpec((_NH, D), lambda i: (i, 0)),
            pl.BlockSpec((1, _NH, D), lambda i: (i, 0, 0)),
            pl.BlockSpec((_NH, 1), lambda i: (i, 0)),
            pl.BlockSpec((D, D), lambda i: (0, 0)),
        ],
        out_specs=pl.BlockSpec((_NH, D), lambda i: (i, 0)),
        out_shape=jax.ShapeDtypeStruct((N, D), jnp.float32),
    )(s_pre, parts, node_attr, W_lin2)


def kernel(node_input, node_attr, edge_src, edge_dst, edge_attr, edge_scalars,
           W_sc, W_lin1, W_fc1, W_fc2, W_lin2):
    wa = _edge_mlp(edge_scalars, edge_attr, W_fc1, W_fc2)
    s_pre, xl = _node_pre(node_input, node_attr, W_sc, W_lin1)
    parts = _sc_scatter(xl, wa, edge_src, edge_dst)
    return _post(s_pre, parts, node_attr, W_lin2)
